# Initial kernel scaffold; baseline (speedup 1.0000x reference)
import jax
import jax.numpy as jnp
from jax.experimental import pallas as pl


def kernel(x, initial_state, Wx_f, Wh_f, b_f, Wx_b, Wh_b, b_b):
    raise NotImplementedError("write your pallas kernel here")


# speedup vs baseline: 3.5071x; 3.5071x over previous
import jax
import jax.numpy as jnp
from jax.experimental import pallas as pl
from jax.experimental.pallas import tpu as pltpu

S, B, I = 512, 64, 1024
H = 512
TS = 16

_INTERPRET = False


def _proj_kernel(x_ref, w_ref, b_ref, out_ref):
    x2 = x_ref[...].reshape(TS * B, I)
    res = jnp.dot(x2, w_ref[0], preferred_element_type=jnp.float32) + b_ref[0]
    for g in range(3):
        out_ref[g] = res[:, g * H:(g + 1) * H].reshape(TS, B, H)


def _rec_kernel(xg_ref, whru_ref, whc_ref, h0_ref, out_ref, state_ref):
    t = pl.program_id(1)

    @pl.when(t == 0)
    def _():
        state_ref[...] = h0_ref[...]

    h = state_ref[...]
    z = jnp.dot(h, whru_ref[0], preferred_element_type=jnp.float32)
    r = jax.nn.sigmoid(xg_ref[0, 0] + z[:, :H])
    u = jax.nn.sigmoid(xg_ref[1, 0] + z[:, H:])
    c = jnp.tanh(xg_ref[2, 0]
                 + jnp.dot(r * h, whc_ref[0], preferred_element_type=jnp.float32))
    h_new = u * h + (1.0 - u) * c
    state_ref[...] = h_new
    out_ref[0] = h_new


def kernel(x, initial_state, Wx_f, Wh_f, b_f, Wx_b, Wh_b, b_b):
    Wx = jnp.stack([
        jnp.transpose(Wx_f, (1, 0, 2)).reshape(I, 3 * H),
        jnp.transpose(Wx_b, (1, 0, 2)).reshape(I, 3 * H),
    ])
    bias = jnp.stack([b_f.reshape(1, 3 * H), b_b.reshape(1, 3 * H)])
    Wh_ru = jnp.stack([
        jnp.concatenate([Wh_f[0], Wh_f[1]], axis=-1),
        jnp.concatenate([Wh_b[0], Wh_b[1]], axis=-1),
    ])
    Wh_c = jnp.stack([Wh_f[2], Wh_b[2]])

    xg = pl.pallas_call(
        _proj_kernel,
        grid=(2, S // TS),
        in_specs=[
            pl.BlockSpec((TS, B, I), lambda d, si: (si, 0, 0)),
            pl.BlockSpec((1, I, 3 * H), lambda d, si: (d, 0, 0)),
            pl.BlockSpec((1, 1, 3 * H), lambda d, si: (d, 0, 0)),
        ],
        out_specs=pl.BlockSpec((3, TS, B, H), lambda d, si: (0, si, 0, d)),
        out_shape=jax.ShapeDtypeStruct((3, S, B, 2 * H), jnp.float32),
        compiler_params=pltpu.CompilerParams(
            dimension_semantics=("parallel", "arbitrary"),
            vmem_limit_bytes=56 * 1024 * 1024,
        ),
        name="bigru_proj",
        interpret=_INTERPRET,
    )(x, Wx, bias)

    def t_eff(d, t):
        return jnp.where(d == 0, t, S - 1 - t)

    out, state = pl.pallas_call(
        _rec_kernel,
        grid=(2, S),
        in_specs=[
            pl.BlockSpec((3, 1, B, H), lambda d, t: (0, t_eff(d, t), 0, d)),
            pl.BlockSpec((1, H, 2 * H), lambda d, t: (d, 0, 0)),
            pl.BlockSpec((1, H, H), lambda d, t: (d, 0, 0)),
            pl.BlockSpec((B, H), lambda d, t: (0, d)),
        ],
        out_specs=[
            pl.BlockSpec((1, B, H), lambda d, t: (t_eff(d, t), 0, d)),
            pl.BlockSpec((B, H), lambda d, t: (0, d)),
        ],
        out_shape=[
            jax.ShapeDtypeStruct((S, B, 2 * H), jnp.float32),
            jax.ShapeDtypeStruct((B, 2 * H), jnp.float32),
        ],
        compiler_params=pltpu.CompilerParams(
            dimension_semantics=("parallel", "arbitrary"),
            vmem_limit_bytes=56 * 1024 * 1024,
        ),
        name="bigru_rec",
        interpret=_INTERPRET,
    )(xg, Wh_ru, Wh_c, initial_state)

    return out, state


# speedup vs baseline: 5.1323x; 1.4634x over previous
import jax
import jax.numpy as jnp
from jax.experimental import pallas as pl
from jax.experimental.pallas import tpu as pltpu

S, B, I = 512, 64, 1024
H = 512
TS = 16
NSLOT = 4

_INTERPRET = False


def _proj_kernel(x_ref, w_ref, b_ref, out_ref):
    x2 = x_ref[...].reshape(TS * B, I).astype(jnp.bfloat16)
    res = jnp.dot(x2, w_ref[...], preferred_element_type=jnp.float32)
    res = res + b_ref[...]
    for g in range(3):
        out_ref[g] = (res[:, g * 2 * H:(g + 1) * 2 * H]
                      .astype(jnp.bfloat16).reshape(TS, B, 2 * H))


def _rec_kernel(xgf_ref, xgb_ref, whru_ref, whc_ref, h0_ref,
                out_ref, state_ref, h_scr, obuf, sems):
    t = pl.program_id(0)
    slot = jax.lax.rem(t, NSLOT)

    @pl.when(t == 0)
    def _():
        h_scr[0] = h0_ref[:, :H]
        h_scr[1] = h0_ref[:, H:]

    hf = h_scr[0]
    hb = h_scr[1]
    zf = jnp.dot(hf.astype(jnp.bfloat16), whru_ref[0],
                 preferred_element_type=jnp.float32)
    zb = jnp.dot(hb.astype(jnp.bfloat16), whru_ref[1],
                 preferred_element_type=jnp.float32)
    rf = jax.nn.sigmoid(xgf_ref[0, 0] + zf[:, :H])
    uf = jax.nn.sigmoid(xgf_ref[1, 0] + zf[:, H:])
    rb = jax.nn.sigmoid(xgb_ref[0, 0] + zb[:, :H])
    ub = jax.nn.sigmoid(xgb_ref[1, 0] + zb[:, H:])
    cf = jnp.tanh(xgf_ref[2, 0]
                  + jnp.dot((rf * hf).astype(jnp.bfloat16), whc_ref[0],
                            preferred_element_type=jnp.float32))
    cb = jnp.tanh(xgb_ref[2, 0]
                  + jnp.dot((rb * hb).astype(jnp.bfloat16), whc_ref[1],
                            preferred_element_type=jnp.float32))
    hf2 = uf * hf + (1.0 - uf) * cf
    hb2 = ub * hb + (1.0 - ub) * cb
    h_scr[0] = hf2
    h_scr[1] = hb2

    @pl.when(t >= NSLOT)
    def _():
        for d in range(2):
            pltpu.make_async_copy(obuf.at[d, slot], obuf.at[d, slot],
                                  sems.at[d, slot]).wait()

    obuf[0, slot] = hf2
    obuf[1, slot] = hb2
    pltpu.make_async_copy(obuf.at[0, slot],
                          out_ref.at[t, :, pl.ds(0, H)],
                          sems.at[0, slot]).start()
    pltpu.make_async_copy(obuf.at[1, slot],
                          out_ref.at[S - 1 - t, :, pl.ds(H, H)],
                          sems.at[1, slot]).start()

    @pl.when(t == S - 1)
    def _():
        state_ref[:, :H] = hf2
        state_ref[:, H:] = hb2
        for d in range(2):
            for s_ in range(NSLOT):
                pltpu.make_async_copy(obuf.at[d, s_], obuf.at[d, s_],
                                      sems.at[d, s_]).wait()


def kernel(x, initial_state, Wx_f, Wh_f, b_f, Wx_b, Wh_b, b_b):
    Wx = (jnp.stack([Wx_f, Wx_b], axis=1)
          .transpose(2, 0, 1, 3).reshape(I, 6 * H).astype(jnp.bfloat16))
    bias = jnp.stack([b_f, b_b], axis=1).reshape(1, 6 * H)
    Wh_ru = jnp.stack([
        jnp.concatenate([Wh_f[0], Wh_f[1]], axis=-1),
        jnp.concatenate([Wh_b[0], Wh_b[1]], axis=-1),
    ]).astype(jnp.bfloat16)
    Wh_c = jnp.stack([Wh_f[2], Wh_b[2]]).astype(jnp.bfloat16)

    xg = pl.pallas_call(
        _proj_kernel,
        grid=(S // TS,),
        in_specs=[
            pl.BlockSpec((TS, B, I), lambda si: (si, 0, 0)),
            pl.BlockSpec((I, 6 * H), lambda si: (0, 0)),
            pl.BlockSpec((1, 6 * H), lambda si: (0, 0)),
        ],
        out_specs=pl.BlockSpec((3, TS, B, 2 * H), lambda si: (0, si, 0, 0)),
        out_shape=jax.ShapeDtypeStruct((3, S, B, 2 * H), jnp.bfloat16),
        compiler_params=pltpu.CompilerParams(
            dimension_semantics=("arbitrary",),
            vmem_limit_bytes=56 * 1024 * 1024,
        ),
        name="bigru_proj",
        interpret=_INTERPRET,
    )(x, Wx, bias)

    out, state = pl.pallas_call(
        _rec_kernel,
        grid=(S,),
        in_specs=[
            pl.BlockSpec((3, 1, B, H), lambda t: (0, t, 0, 0)),
            pl.BlockSpec((3, 1, B, H), lambda t: (0, S - 1 - t, 0, 1)),
            pl.BlockSpec((2, H, 2 * H), lambda t: (0, 0, 0)),
            pl.BlockSpec((2, H, H), lambda t: (0, 0, 0)),
            pl.BlockSpec((B, 2 * H), lambda t: (0, 0)),
        ],
        out_specs=[
            pl.BlockSpec(memory_space=pl.ANY),
            pl.BlockSpec((B, 2 * H), lambda t: (0, 0)),
        ],
        out_shape=[
            jax.ShapeDtypeStruct((S, B, 2 * H), jnp.float32),
            jax.ShapeDtypeStruct((B, 2 * H), jnp.float32),
        ],
        scratch_shapes=[
            pltpu.VMEM((2, B, H), jnp.float32),
            pltpu.VMEM((2, NSLOT, B, H), jnp.float32),
            pltpu.SemaphoreType.DMA((2, NSLOT)),
        ],
        compiler_params=pltpu.CompilerParams(
            dimension_semantics=("arbitrary",),
            vmem_limit_bytes=56 * 1024 * 1024,
        ),
        name="bigru_rec",
        interpret=_INTERPRET,
    )(xg, xg, Wh_ru, Wh_c, initial_state)

    return out, state


# speedup vs baseline: 6.0014x; 1.1693x over previous
import jax
import jax.numpy as jnp
from jax.experimental import pallas as pl
from jax.experimental.pallas import tpu as pltpu

S, B, I = 512, 64, 1024
H = 512
TS = 16
NSLOT = 4
U = 2

_INTERPRET = False


def _proj_kernel(x_ref, w_ref, b_ref, out_ref):
    x2 = x_ref[...].reshape(TS * B, I).astype(jnp.bfloat16)
    res = jnp.dot(x2, w_ref[...], preferred_element_type=jnp.float32)
    res = res + b_ref[...]
    for g in range(3):
        out_ref[g] = (res[:, g * 2 * H:(g + 1) * 2 * H]
                      .astype(jnp.bfloat16).reshape(TS, B, 2 * H))


def _gru_step(h, xt, whru, whc):
    z = jnp.dot(h.astype(jnp.bfloat16), whru,
                preferred_element_type=jnp.float32)
    r = jax.nn.sigmoid(xt[0] + z[:, :H])
    u = jax.nn.sigmoid(xt[1] + z[:, H:])
    c = jnp.tanh(xt[2] + jnp.dot((r * h).astype(jnp.bfloat16), whc,
                                 preferred_element_type=jnp.float32))
    return u * h + (1.0 - u) * c


def _rec_kernel(xgf_ref, xgb_ref, whru_ref, whc_ref, h0_ref,
                out_ref, state_ref, h_scr, obuf, sems):
    i = pl.program_id(0)
    slot = jax.lax.rem(i, NSLOT)

    @pl.when(i == 0)
    def _():
        h_scr[0] = h0_ref[:, :H]
        h_scr[1] = h0_ref[:, H:]

    @pl.when(i >= NSLOT)
    def _():
        for d in range(2):
            pltpu.make_async_copy(obuf.at[d, slot], obuf.at[d, slot],
                                  sems.at[d, slot]).wait()

    hf = h_scr[0]
    hb = h_scr[1]
    for k in range(U):
        hf = _gru_step(hf, [xgf_ref[g, k] for g in range(3)],
                       whru_ref[0], whc_ref[0])
        hb = _gru_step(hb, [xgb_ref[g, U - 1 - k] for g in range(3)],
                       whru_ref[1], whc_ref[1])
        obuf[0, slot, k] = hf
        obuf[1, slot, U - 1 - k] = hb
    h_scr[0] = hf
    h_scr[1] = hb

    pltpu.make_async_copy(obuf.at[0, slot],
                          out_ref.at[pl.ds(U * i, U), :, pl.ds(0, H)],
                          sems.at[0, slot]).start()
    pltpu.make_async_copy(obuf.at[1, slot],
                          out_ref.at[pl.ds(S - U - U * i, U), :, pl.ds(H, H)],
                          sems.at[1, slot]).start()

    @pl.when(i == S // U - 1)
    def _():
        state_ref[:, :H] = hf
        state_ref[:, H:] = hb
        for d in range(2):
            for s_ in range(NSLOT):
                pltpu.make_async_copy(obuf.at[d, s_], obuf.at[d, s_],
                                      sems.at[d, s_]).wait()


def kernel(x, initial_state, Wx_f, Wh_f, b_f, Wx_b, Wh_b, b_b):
    Wx = (jnp.stack([Wx_f, Wx_b], axis=1)
          .transpose(2, 0, 1, 3).reshape(I, 6 * H).astype(jnp.bfloat16))
    bias = jnp.stack([b_f, b_b], axis=1).reshape(1, 6 * H)
    Wh_ru = jnp.stack([
        jnp.concatenate([Wh_f[0], Wh_f[1]], axis=-1),
        jnp.concatenate([Wh_b[0], Wh_b[1]], axis=-1),
    ]).astype(jnp.bfloat16)
    Wh_c = jnp.stack([Wh_f[2], Wh_b[2]]).astype(jnp.bfloat16)

    xg = pl.pallas_call(
        _proj_kernel,
        grid=(S // TS,),
        in_specs=[
            pl.BlockSpec((TS, B, I), lambda si: (si, 0, 0)),
            pl.BlockSpec((I, 6 * H), lambda si: (0, 0)),
            pl.BlockSpec((1, 6 * H), lambda si: (0, 0)),
        ],
        out_specs=pl.BlockSpec((3, TS, B, 2 * H), lambda si: (0, si, 0, 0)),
        out_shape=jax.ShapeDtypeStruct((3, S, B, 2 * H), jnp.bfloat16),
        compiler_params=pltpu.CompilerParams(
            dimension_semantics=("arbitrary",),
            vmem_limit_bytes=56 * 1024 * 1024,
        ),
        name="bigru_proj",
        interpret=_INTERPRET,
    )(x, Wx, bias)

    out, state = pl.pallas_call(
        _rec_kernel,
        grid=(S // U,),
        in_specs=[
            pl.BlockSpec((3, U, B, H), lambda i: (0, i, 0, 0)),
            pl.BlockSpec((3, U, B, H), lambda i: (0, S // U - 1 - i, 0, 1)),
            pl.BlockSpec((2, H, 2 * H), lambda i: (0, 0, 0)),
            pl.BlockSpec((2, H, H), lambda i: (0, 0, 0)),
            pl.BlockSpec((B, 2 * H), lambda i: (0, 0)),
        ],
        out_specs=[
            pl.BlockSpec(memory_space=pl.ANY),
            pl.BlockSpec((B, 2 * H), lambda i: (0, 0)),
        ],
        out_shape=[
            jax.ShapeDtypeStruct((S, B, 2 * H), jnp.float32),
            jax.ShapeDtypeStruct((B, 2 * H), jnp.float32),
        ],
        scratch_shapes=[
            pltpu.VMEM((2, B, H), jnp.float32),
            pltpu.VMEM((2, NSLOT, U, B, H), jnp.float32),
            pltpu.SemaphoreType.DMA((2, NSLOT)),
        ],
        compiler_params=pltpu.CompilerParams(
            dimension_semantics=("arbitrary",),
            vmem_limit_bytes=56 * 1024 * 1024,
        ),
        name="bigru_rec",
        interpret=_INTERPRET,
    )(xg, xg, Wh_ru, Wh_c, initial_state)

    return out, state


# speedup vs baseline: 6.2755x; 1.0457x over previous
import jax
import jax.numpy as jnp
from jax.experimental import pallas as pl
from jax.experimental.pallas import tpu as pltpu

S, B, I = 512, 64, 1024
H = 512
TS = 16
NSLOT = 4
U = 4

_INTERPRET = False


def _proj_kernel(x_ref, w_ref, b_ref, out_ref):
    x2 = x_ref[...].reshape(TS * B, I).astype(jnp.bfloat16)
    res = jnp.dot(x2, w_ref[...], preferred_element_type=jnp.float32)
    res = res + b_ref[...]
    for g in range(3):
        out_ref[g] = (res[:, g * 2 * H:(g + 1) * 2 * H]
                      .astype(jnp.bfloat16).reshape(TS, B, 2 * H))


def _gru_step(h, xt, whru, whc):
    z = jnp.dot(h.astype(jnp.bfloat16), whru,
                preferred_element_type=jnp.float32)
    r = jax.nn.sigmoid(xt[0] + z[:, :H])
    u = jax.nn.sigmoid(xt[1] + z[:, H:])
    c = jnp.tanh(xt[2] + jnp.dot((r * h).astype(jnp.bfloat16), whc,
                                 preferred_element_type=jnp.float32))
    return u * h + (1.0 - u) * c


def _rec_kernel(xgf_ref, xgb_ref, whru_ref, whc_ref, h0_ref,
                out_ref, state_ref, h_scr, obuf, sems):
    i = pl.program_id(0)
    slot = jax.lax.rem(i, NSLOT)

    @pl.when(i == 0)
    def _():
        h_scr[0] = h0_ref[:, :H]
        h_scr[1] = h0_ref[:, H:]

    @pl.when(i >= NSLOT)
    def _():
        for d in range(2):
            pltpu.make_async_copy(obuf.at[d, slot], obuf.at[d, slot],
                                  sems.at[d, slot]).wait()

    hf = h_scr[0]
    hb = h_scr[1]
    for k in range(U):
        hf = _gru_step(hf, [xgf_ref[g, k] for g in range(3)],
                       whru_ref[0], whc_ref[0])
        hb = _gru_step(hb, [xgb_ref[g, U - 1 - k] for g in range(3)],
                       whru_ref[1], whc_ref[1])
        obuf[0, slot, k] = hf
        obuf[1, slot, U - 1 - k] = hb
    h_scr[0] = hf
    h_scr[1] = hb

    pltpu.make_async_copy(obuf.at[0, slot],
                          out_ref.at[pl.ds(U * i, U), :, pl.ds(0, H)],
                          sems.at[0, slot]).start()
    pltpu.make_async_copy(obuf.at[1, slot],
                          out_ref.at[pl.ds(S - U - U * i, U), :, pl.ds(H, H)],
                          sems.at[1, slot]).start()

    @pl.when(i == S // U - 1)
    def _():
        state_ref[:, :H] = hf
        state_ref[:, H:] = hb
        for d in range(2):
            for s_ in range(NSLOT):
                pltpu.make_async_copy(obuf.at[d, s_], obuf.at[d, s_],
                                      sems.at[d, s_]).wait()


def kernel(x, initial_state, Wx_f, Wh_f, b_f, Wx_b, Wh_b, b_b):
    Wx = (jnp.stack([Wx_f, Wx_b], axis=1)
          .transpose(2, 0, 1, 3).reshape(I, 6 * H).astype(jnp.bfloat16))
    bias = jnp.stack([b_f, b_b], axis=1).reshape(1, 6 * H)
    Wh_ru = jnp.stack([
        jnp.concatenate([Wh_f[0], Wh_f[1]], axis=-1),
        jnp.concatenate([Wh_b[0], Wh_b[1]], axis=-1),
    ]).astype(jnp.bfloat16)
    Wh_c = jnp.stack([Wh_f[2], Wh_b[2]]).astype(jnp.bfloat16)

    xg = pl.pallas_call(
        _proj_kernel,
        grid=(S // TS,),
        in_specs=[
            pl.BlockSpec((TS, B, I), lambda si: (si, 0, 0)),
            pl.BlockSpec((I, 6 * H), lambda si: (0, 0)),
            pl.BlockSpec((1, 6 * H), lambda si: (0, 0)),
        ],
        out_specs=pl.BlockSpec((3, TS, B, 2 * H), lambda si: (0, si, 0, 0)),
        out_shape=jax.ShapeDtypeStruct((3, S, B, 2 * H), jnp.bfloat16),
        compiler_params=pltpu.CompilerParams(
            dimension_semantics=("arbitrary",),
            vmem_limit_bytes=56 * 1024 * 1024,
        ),
        name="bigru_proj",
        interpret=_INTERPRET,
    )(x, Wx, bias)

    out, state = pl.pallas_call(
        _rec_kernel,
        grid=(S // U,),
        in_specs=[
            pl.BlockSpec((3, U, B, H), lambda i: (0, i, 0, 0)),
            pl.BlockSpec((3, U, B, H), lambda i: (0, S // U - 1 - i, 0, 1)),
            pl.BlockSpec((2, H, 2 * H), lambda i: (0, 0, 0)),
            pl.BlockSpec((2, H, H), lambda i: (0, 0, 0)),
            pl.BlockSpec((B, 2 * H), lambda i: (0, 0)),
        ],
        out_specs=[
            pl.BlockSpec(memory_space=pl.ANY),
            pl.BlockSpec((B, 2 * H), lambda i: (0, 0)),
        ],
        out_shape=[
            jax.ShapeDtypeStruct((S, B, 2 * H), jnp.float32),
            jax.ShapeDtypeStruct((B, 2 * H), jnp.float32),
        ],
        scratch_shapes=[
            pltpu.VMEM((2, B, H), jnp.float32),
            pltpu.VMEM((2, NSLOT, U, B, H), jnp.float32),
            pltpu.SemaphoreType.DMA((2, NSLOT)),
        ],
        compiler_params=pltpu.CompilerParams(
            dimension_semantics=("arbitrary",),
            vmem_limit_bytes=56 * 1024 * 1024,
        ),
        name="bigru_rec",
        interpret=_INTERPRET,
    )(xg, xg, Wh_ru, Wh_c, initial_state)

    return out, state


# speedup vs baseline: 6.8411x; 1.0901x over previous
import jax
import jax.numpy as jnp
from jax.experimental import pallas as pl
from jax.experimental.pallas import tpu as pltpu

S, B, I = 512, 64, 1024
H = 512
TS = 16
NSLOT = 4
U = 4

_INTERPRET = False


def _proj_kernel(x_ref, w_ref, b_ref, out_ref):
    x2 = x_ref[...].reshape(TS * B, I).astype(jnp.bfloat16)
    for g in range(3):
        for d in range(2):
            res = jnp.dot(x2, w_ref[d, g], preferred_element_type=jnp.float32)
            res = res + b_ref[0, (2 * g + d) * H:(2 * g + d + 1) * H]
            out_ref[g, :, :, d * H:(d + 1) * H] = (
                res.astype(jnp.bfloat16).reshape(TS, B, H))


def _gru_step(h, xt, whru, whc):
    z = jnp.dot(h.astype(jnp.bfloat16), whru,
                preferred_element_type=jnp.float32)
    r = jax.nn.sigmoid(xt[0] + z[:, :H])
    u = jax.nn.sigmoid(xt[1] + z[:, H:])
    c = jnp.tanh(xt[2] + jnp.dot((r * h).astype(jnp.bfloat16), whc,
                                 preferred_element_type=jnp.float32))
    return u * h + (1.0 - u) * c


def _rec_kernel(xgf_ref, xgb_ref, whru_ref, whc_ref, h0_ref,
                out_ref, state_ref, h_scr, obuf, sems):
    i = pl.program_id(0)
    slot = jax.lax.rem(i, NSLOT)

    @pl.when(i == 0)
    def _():
        h_scr[0] = h0_ref[:, :H]
        h_scr[1] = h0_ref[:, H:]

    @pl.when(i >= NSLOT)
    def _():
        for d in range(2):
            pltpu.make_async_copy(obuf.at[d, slot], obuf.at[d, slot],
                                  sems.at[d, slot]).wait()

    hf = h_scr[0]
    hb = h_scr[1]
    for k in range(U):
        hf = _gru_step(hf, [xgf_ref[g, k] for g in range(3)],
                       whru_ref[0], whc_ref[0])
        hb = _gru_step(hb, [xgb_ref[g, U - 1 - k] for g in range(3)],
                       whru_ref[1], whc_ref[1])
        obuf[0, slot, k] = hf
        obuf[1, slot, U - 1 - k] = hb
    h_scr[0] = hf
    h_scr[1] = hb

    pltpu.make_async_copy(obuf.at[0, slot],
                          out_ref.at[pl.ds(U * i, U), :, pl.ds(0, H)],
                          sems.at[0, slot]).start()
    pltpu.make_async_copy(obuf.at[1, slot],
                          out_ref.at[pl.ds(S - U - U * i, U), :, pl.ds(H, H)],
                          sems.at[1, slot]).start()

    @pl.when(i == S // U - 1)
    def _():
        state_ref[:, :H] = hf
        state_ref[:, H:] = hb
        for d in range(2):
            for s_ in range(NSLOT):
                pltpu.make_async_copy(obuf.at[d, s_], obuf.at[d, s_],
                                      sems.at[d, s_]).wait()


def kernel(x, initial_state, Wx_f, Wh_f, b_f, Wx_b, Wh_b, b_b):
    Wx = jnp.stack([Wx_f, Wx_b]).astype(jnp.bfloat16)
    bias = jnp.stack([b_f, b_b], axis=1).reshape(1, 6 * H)
    Wh_ru = jnp.stack([
        jnp.concatenate([Wh_f[0], Wh_f[1]], axis=-1),
        jnp.concatenate([Wh_b[0], Wh_b[1]], axis=-1),
    ]).astype(jnp.bfloat16)
    Wh_c = jnp.stack([Wh_f[2], Wh_b[2]]).astype(jnp.bfloat16)

    xg = pl.pallas_call(
        _proj_kernel,
        grid=(S // TS,),
        in_specs=[
            pl.BlockSpec((TS, B, I), lambda si: (si, 0, 0)),
            pl.BlockSpec((2, 3, I, H), lambda si: (0, 0, 0, 0)),
            pl.BlockSpec((1, 6 * H), lambda si: (0, 0)),
        ],
        out_specs=pl.BlockSpec((3, TS, B, 2 * H), lambda si: (0, si, 0, 0)),
        out_shape=jax.ShapeDtypeStruct((3, S, B, 2 * H), jnp.bfloat16),
        compiler_params=pltpu.CompilerParams(
            dimension_semantics=("arbitrary",),
            vmem_limit_bytes=56 * 1024 * 1024,
        ),
        name="bigru_proj",
        interpret=_INTERPRET,
    )(x, Wx, bias)

    out, state = pl.pallas_call(
        _rec_kernel,
        grid=(S // U,),
        in_specs=[
            pl.BlockSpec((3, U, B, H), lambda i: (0, i, 0, 0)),
            pl.BlockSpec((3, U, B, H), lambda i: (0, S // U - 1 - i, 0, 1)),
            pl.BlockSpec((2, H, 2 * H), lambda i: (0, 0, 0)),
            pl.BlockSpec((2, H, H), lambda i: (0, 0, 0)),
            pl.BlockSpec((B, 2 * H), lambda i: (0, 0)),
        ],
        out_specs=[
            pl.BlockSpec(memory_space=pl.ANY),
            pl.BlockSpec((B, 2 * H), lambda i: (0, 0)),
        ],
        out_shape=[
            jax.ShapeDtypeStruct((S, B, 2 * H), jnp.float32),
            jax.ShapeDtypeStruct((B, 2 * H), jnp.float32),
        ],
        scratch_shapes=[
            pltpu.VMEM((2, B, H), jnp.float32),
            pltpu.VMEM((2, NSLOT, U, B, H), jnp.float32),
            pltpu.SemaphoreType.DMA((2, NSLOT)),
        ],
        compiler_params=pltpu.CompilerParams(
            dimension_semantics=("arbitrary",),
            vmem_limit_bytes=56 * 1024 * 1024,
        ),
        name="bigru_rec",
        interpret=_INTERPRET,
    )(xg, xg, Wh_ru, Wh_c, initial_state)

    return out, state
